# baseline (device time: 19342 ns/iter reference)
import jax
import jax.numpy as jnp
from jax import lax
from jax.experimental import pallas as pl
from jax.experimental.pallas import tpu as pltpu

N_DEV = 4
B, Sq, Skv, Hq, Dh = 2, 128, 128, 16, 64
H_LOC = Hq // N_DEV
D_LOC = H_LOC * Dh
D_MODEL = 512
NEG_INF = -1e9


def _body(x_ref, wq_ref, k_hbm, v_hbm, wo_ref, out_ref,
          k_loc, v_loc, comm_ref, local_sems, send_sems, recv_sems):
    my_pos = lax.axis_index("i")
    h0 = my_pos * H_LOC
    p_a = my_pos ^ 1
    p_b = (N_DEV - 1) - my_pos

    kv_copies = []
    for b in range(B):
        kc = pltpu.make_async_copy(
            k_hbm.at[b, :, pl.ds(h0, H_LOC), :], k_loc.at[b],
            local_sems.at[2 * b])
        vc = pltpu.make_async_copy(
            v_hbm.at[b, :, pl.ds(h0, H_LOC), :], v_loc.at[b],
            local_sems.at[2 * b + 1])
        kc.start()
        vc.start()
        kv_copies.append((kc, vc))

    barrier_sem = pltpu.get_barrier_semaphore()
    for nbr in (p_a, p_b):
        pl.semaphore_signal(
            barrier_sem, inc=1,
            device_id=(nbr,), device_id_type=pl.DeviceIdType.MESH,
        )
    pl.semaphore_wait(barrier_sem, 2)

    qb = lax.broadcasted_iota(jnp.int32, (Sq, Skv), 0) // 64
    kb = lax.broadcasted_iota(jnp.int32, (Sq, Skv), 1) // 64
    mask = (qb == kb) | (kb == 0) | (lax.rem(qb + kb, 3) == 0)

    def partial_for_batch(b):
        q_b = jnp.dot(x_ref[b], wq_ref[...],
                      preferred_element_type=jnp.float32) * 0.125
        kc, vc = kv_copies[b]
        kc.wait()
        vc.wait()
        ctx_h = []
        for h in range(H_LOC):
            q = q_b[:, h * Dh:(h + 1) * Dh]
            k = k_loc[b, :, h, :]
            s = lax.dot_general(
                q, k, (((1,), (1,)), ((), ())),
                preferred_element_type=jnp.float32)
            e = jnp.where(mask, jnp.exp(s), 0.0)
            w = e / jnp.sum(e, axis=-1, keepdims=True)
            ctx_h.append(jnp.dot(w, v_loc[b, :, h, :],
                                 preferred_element_type=jnp.float32))
        ctx_b = jnp.concatenate(ctx_h, axis=1)
        return jnp.dot(ctx_b, wo_ref[...],
                       preferred_element_type=jnp.float32)

    def exchange(half, partner, slot):
        return pltpu.make_async_remote_copy(
            src_ref=out_ref.at[half],
            dst_ref=comm_ref.at[slot],
            send_sem=send_sems.at[slot],
            recv_sem=recv_sems.at[slot],
            device_id=(partner,),
            device_id_type=pl.DeviceIdType.MESH,
        )

    out_ref[0] = partial_for_batch(0)
    r1h0 = exchange(0, p_a, 0)
    r1h0.start()

    out_ref[1] = partial_for_batch(1)
    r1h1 = exchange(1, p_b, 1)
    r1h1.start()

    r1h0.wait()
    out_ref[0] = out_ref[0] + comm_ref[0]
    r2h0 = exchange(0, p_b, 2)
    r2h0.start()

    r1h1.wait()
    out_ref[1] = out_ref[1] + comm_ref[1]
    r2h1 = exchange(1, p_a, 3)
    r2h1.start()

    r2h0.wait()
    out_ref[0] = out_ref[0] + comm_ref[2]
    r2h1.wait()
    out_ref[1] = out_ref[1] + comm_ref[3]


def kernel(x, Wq, K_ext, V_ext, Wo):
    return pl.pallas_call(
        _body,
        out_shape=jax.ShapeDtypeStruct((B, Sq, D_MODEL), jnp.float32),
        in_specs=[
            pl.BlockSpec(memory_space=pltpu.VMEM),
            pl.BlockSpec(memory_space=pltpu.VMEM),
            pl.BlockSpec(memory_space=pltpu.MemorySpace.HBM),
            pl.BlockSpec(memory_space=pltpu.MemorySpace.HBM),
            pl.BlockSpec(memory_space=pltpu.VMEM),
        ],
        out_specs=pl.BlockSpec(memory_space=pltpu.VMEM),
        scratch_shapes=[
            pltpu.VMEM((B, Skv, H_LOC, Dh), jnp.float32),
            pltpu.VMEM((B, Skv, H_LOC, Dh), jnp.float32),
            pltpu.VMEM((4, Sq, D_MODEL), jnp.float32),
            pltpu.SemaphoreType.DMA((4,)),
            pltpu.SemaphoreType.DMA((4,)),
            pltpu.SemaphoreType.DMA((4,)),
        ],
        compiler_params=pltpu.CompilerParams(collective_id=0),
    )(x, Wq, K_ext, V_ext, Wo)


# device time: 15011 ns/iter; 1.2885x vs baseline; 1.2885x over previous
import jax
import jax.numpy as jnp
from jax import lax
from jax.experimental import pallas as pl
from jax.experimental.pallas import tpu as pltpu

N_DEV = 4
B, Sq, Skv, Hq, Dh = 2, 128, 128, 16, 64
H_LOC = Hq // N_DEV
D_LOC = H_LOC * Dh
D_MODEL = 512
NEG_INF = -1e9


def _body(x_ref, wq_ref, k_hbm, v_hbm, wo_ref, out_ref,
          k_loc, v_loc, send_ref, comm_ref, local_sems, send_sems, recv_sems):
    my_pos = lax.axis_index("i")
    h0 = my_pos * H_LOC
    p_a = my_pos ^ 1
    p_b = (N_DEV - 1) - my_pos

    kv_copies = []
    for b in range(B):
        kc = pltpu.make_async_copy(
            k_hbm.at[b, :, pl.ds(h0, H_LOC), :], k_loc.at[b],
            local_sems.at[2 * b])
        vc = pltpu.make_async_copy(
            v_hbm.at[b, :, pl.ds(h0, H_LOC), :], v_loc.at[b],
            local_sems.at[2 * b + 1])
        kc.start()
        vc.start()
        kv_copies.append((kc, vc))

    barrier_sem = pltpu.get_barrier_semaphore()
    for nbr in (p_a, p_b):
        pl.semaphore_signal(
            barrier_sem, inc=1,
            device_id=(nbr,), device_id_type=pl.DeviceIdType.MESH,
        )

    qb = lax.broadcasted_iota(jnp.int32, (Sq, Skv), 0) // 64
    kb = lax.broadcasted_iota(jnp.int32, (Sq, Skv), 1) // 64
    mask = (qb == kb) | (kb == 0) | (lax.rem(qb + kb, 3) == 0)

    def partial_for_batch(b):
        q_b = jnp.dot(x_ref[b], wq_ref[...],
                      preferred_element_type=jnp.float32) * 0.125
        kc, vc = kv_copies[b]
        kc.wait()
        vc.wait()
        ctx_h = []
        for h in range(H_LOC):
            q = q_b[:, h * Dh:(h + 1) * Dh]
            k = k_loc[b, :, h, :]
            s = lax.dot_general(
                q, k, (((1,), (1,)), ((), ())),
                preferred_element_type=jnp.float32)
            e = jnp.where(mask, jnp.exp(s), 0.0)
            r = 1.0 / jnp.sum(e, axis=-1, keepdims=True)
            ctx_h.append(jnp.dot(e, v_loc[b, :, h, :],
                                 preferred_element_type=jnp.float32) * r)
        ctx_b = jnp.concatenate(ctx_h, axis=1)
        return jnp.dot(ctx_b, wo_ref[...],
                       preferred_element_type=jnp.float32)

    def exchange(slot, partner):
        return pltpu.make_async_remote_copy(
            src_ref=send_ref.at[slot],
            dst_ref=comm_ref.at[slot],
            send_sem=send_sems.at[slot],
            recv_sem=recv_sems.at[slot],
            device_id=(partner,),
            device_id_type=pl.DeviceIdType.MESH,
        )

    C = Sq // 2
    p0 = partial_for_batch(0)
    out_ref[0] = p0
    send_ref[0] = p0[0:C, :].astype(jnp.bfloat16)
    send_ref[1] = p0[C:2 * C, :].astype(jnp.bfloat16)
    pl.semaphore_wait(barrier_sem, 2)
    r1h0 = [exchange(0, p_a), exchange(1, p_a)]
    r1h0[0].start()
    r1h0[1].start()

    p1 = partial_for_batch(1)
    out_ref[1] = p1
    send_ref[2] = p1[0:C, :].astype(jnp.bfloat16)
    send_ref[3] = p1[C:2 * C, :].astype(jnp.bfloat16)
    r1h1 = [exchange(2, p_b), exchange(3, p_b)]
    r1h1[0].start()
    r1h1[1].start()

    r2h0 = [exchange(4, p_b), exchange(5, p_b)]
    r2h1 = [exchange(6, p_a), exchange(7, p_a)]
    for c in range(2):
        r1h0[c].wait()
        s = out_ref[0, c * C:(c + 1) * C, :] + comm_ref[c].astype(jnp.float32)
        out_ref[0, c * C:(c + 1) * C, :] = s
        send_ref[4 + c] = s.astype(jnp.bfloat16)
        r2h0[c].start()
    for c in range(2):
        r1h1[c].wait()
        s = out_ref[1, c * C:(c + 1) * C, :] + comm_ref[2 + c].astype(jnp.float32)
        out_ref[1, c * C:(c + 1) * C, :] = s
        send_ref[6 + c] = s.astype(jnp.bfloat16)
        r2h1[c].start()

    for c in range(2):
        r2h0[c].wait()
        out_ref[0, c * C:(c + 1) * C, :] = (
            out_ref[0, c * C:(c + 1) * C, :]
            + comm_ref[4 + c].astype(jnp.float32))
    for c in range(2):
        r2h1[c].wait()
        out_ref[1, c * C:(c + 1) * C, :] = (
            out_ref[1, c * C:(c + 1) * C, :]
            + comm_ref[6 + c].astype(jnp.float32))


def kernel(x, Wq, K_ext, V_ext, Wo):
    return pl.pallas_call(
        _body,
        out_shape=jax.ShapeDtypeStruct((B, Sq, D_MODEL), jnp.float32),
        in_specs=[
            pl.BlockSpec(memory_space=pltpu.VMEM),
            pl.BlockSpec(memory_space=pltpu.VMEM),
            pl.BlockSpec(memory_space=pltpu.MemorySpace.HBM),
            pl.BlockSpec(memory_space=pltpu.MemorySpace.HBM),
            pl.BlockSpec(memory_space=pltpu.VMEM),
        ],
        out_specs=pl.BlockSpec(memory_space=pltpu.VMEM),
        scratch_shapes=[
            pltpu.VMEM((B, Skv, H_LOC, Dh), jnp.float32),
            pltpu.VMEM((B, Skv, H_LOC, Dh), jnp.float32),
            pltpu.VMEM((8, Sq // 2, D_MODEL), jnp.bfloat16),
            pltpu.VMEM((8, Sq // 2, D_MODEL), jnp.bfloat16),
            pltpu.SemaphoreType.DMA((4,)),
            pltpu.SemaphoreType.DMA((8,)),
            pltpu.SemaphoreType.DMA((8,)),
        ],
        compiler_params=pltpu.CompilerParams(collective_id=0),
    )(x, Wq, K_ext, V_ext, Wo)


# device time: 14974 ns/iter; 1.2917x vs baseline; 1.0025x over previous
import jax
import jax.numpy as jnp
from jax import lax
from jax.experimental import pallas as pl
from jax.experimental.pallas import tpu as pltpu

N_DEV = 4
B, Sq, Skv, Hq, Dh = 2, 128, 128, 16, 64
H_LOC = Hq // N_DEV
D_LOC = H_LOC * Dh
D_MODEL = 512
NEG_INF = -1e9


def _body(x_ref, wq_ref, k_hbm, v_hbm, wo_ref, out_ref,
          k_loc, v_loc, send_ref, comm_ref, local_sems, send_sems, recv_sems):
    my_pos = lax.axis_index("i")
    h0 = my_pos * H_LOC
    p_a = my_pos ^ 1
    p_b = (N_DEV - 1) - my_pos

    kv_copies = []
    for b in range(B):
        kc = pltpu.make_async_copy(
            k_hbm.at[b, :, pl.ds(h0, H_LOC), :], k_loc.at[b],
            local_sems.at[2 * b])
        vc = pltpu.make_async_copy(
            v_hbm.at[b, :, pl.ds(h0, H_LOC), :], v_loc.at[b],
            local_sems.at[2 * b + 1])
        kc.start()
        vc.start()
        kv_copies.append((kc, vc))

    barrier_sem = pltpu.get_barrier_semaphore()
    for nbr in (p_a, p_b):
        pl.semaphore_signal(
            barrier_sem, inc=1,
            device_id=(nbr,), device_id_type=pl.DeviceIdType.MESH,
        )

    qb = lax.broadcasted_iota(jnp.int32, (Sq, Skv), 0) // 64
    kb = lax.broadcasted_iota(jnp.int32, (Sq, Skv), 1) // 64
    mask = (qb == kb) | (kb == 0) | (lax.rem(qb + kb, 3) == 0)

    def partial_for_batch(b):
        q_b = jnp.dot(x_ref[b], wq_ref[...],
                      preferred_element_type=jnp.float32) * 0.125
        kc, vc = kv_copies[b]
        kc.wait()
        vc.wait()
        ctx_h = []
        for h in range(H_LOC):
            q = q_b[:, h * Dh:(h + 1) * Dh]
            k = k_loc[b, :, h, :]
            s = lax.dot_general(
                q, k, (((1,), (1,)), ((), ())),
                preferred_element_type=jnp.float32)
            e = jnp.where(mask, jnp.exp(s), 0.0)
            r = 1.0 / jnp.sum(e, axis=-1, keepdims=True)
            ctx_h.append(jnp.dot(e, v_loc[b, :, h, :],
                                 preferred_element_type=jnp.float32) * r)
        ctx_b = jnp.concatenate(ctx_h, axis=1)
        return jnp.dot(ctx_b, wo_ref[...],
                       preferred_element_type=jnp.float32)

    def exchange(slot, partner):
        return pltpu.make_async_remote_copy(
            src_ref=send_ref.at[slot],
            dst_ref=comm_ref.at[slot],
            send_sem=send_sems.at[slot],
            recv_sem=recv_sems.at[slot],
            device_id=(partner,),
            device_id_type=pl.DeviceIdType.MESH,
        )

    C = Sq // 2
    p0 = partial_for_batch(0)
    send_ref[0] = p0[0:C, :].astype(jnp.bfloat16)
    send_ref[1] = p0[C:2 * C, :].astype(jnp.bfloat16)
    pl.semaphore_wait(barrier_sem, 2)
    r1h0 = [exchange(0, p_a), exchange(1, p_a)]
    r1h0[0].start()
    r1h0[1].start()

    p1 = partial_for_batch(1)
    send_ref[2] = p1[0:C, :].astype(jnp.bfloat16)
    send_ref[3] = p1[C:2 * C, :].astype(jnp.bfloat16)
    r1h1 = [exchange(2, p_b), exchange(3, p_b)]
    r1h1[0].start()
    r1h1[1].start()

    r2h0 = [exchange(4, p_b), exchange(5, p_b)]
    r2h1 = [exchange(6, p_a), exchange(7, p_a)]
    s0 = [None, None]
    s1 = [None, None]
    for c in range(2):
        r1h0[c].wait()
        s0[c] = p0[c * C:(c + 1) * C, :] + comm_ref[c].astype(jnp.float32)
        send_ref[4 + c] = s0[c].astype(jnp.bfloat16)
        r2h0[c].start()
    for c in range(2):
        r1h1[c].wait()
        s1[c] = p1[c * C:(c + 1) * C, :] + comm_ref[2 + c].astype(jnp.float32)
        send_ref[6 + c] = s1[c].astype(jnp.bfloat16)
        r2h1[c].start()

    for c in range(2):
        r2h0[c].wait()
        out_ref[0, c * C:(c + 1) * C, :] = (
            s0[c] + comm_ref[4 + c].astype(jnp.float32))
    for c in range(2):
        r2h1[c].wait()
        out_ref[1, c * C:(c + 1) * C, :] = (
            s1[c] + comm_ref[6 + c].astype(jnp.float32))


def kernel(x, Wq, K_ext, V_ext, Wo):
    return pl.pallas_call(
        _body,
        out_shape=jax.ShapeDtypeStruct((B, Sq, D_MODEL), jnp.float32),
        in_specs=[
            pl.BlockSpec(memory_space=pltpu.VMEM),
            pl.BlockSpec(memory_space=pltpu.VMEM),
            pl.BlockSpec(memory_space=pltpu.MemorySpace.HBM),
            pl.BlockSpec(memory_space=pltpu.MemorySpace.HBM),
            pl.BlockSpec(memory_space=pltpu.VMEM),
        ],
        out_specs=pl.BlockSpec(memory_space=pltpu.VMEM),
        scratch_shapes=[
            pltpu.VMEM((B, Skv, H_LOC, Dh), jnp.float32),
            pltpu.VMEM((B, Skv, H_LOC, Dh), jnp.float32),
            pltpu.VMEM((8, Sq // 2, D_MODEL), jnp.bfloat16),
            pltpu.VMEM((8, Sq // 2, D_MODEL), jnp.bfloat16),
            pltpu.SemaphoreType.DMA((4,)),
            pltpu.SemaphoreType.DMA((8,)),
            pltpu.SemaphoreType.DMA((8,)),
        ],
        compiler_params=pltpu.CompilerParams(collective_id=0),
    )(x, Wq, K_ext, V_ext, Wo)
